# baseline (speedup 1.0000x reference)
import functools
import jax
import jax.numpy as jnp
from jax.experimental import pallas as pl

_C = 81
_TP = 2000


def _phase1_body(ct_ref, conf_ref, lt_ref, ld_ref, ce_ref, ll_ref):
    conf = conf_ref[0]
    ct = ct_ref[0]
    m = jnp.max(conf, axis=-1, keepdims=True)
    e = jnp.exp(conf - m)
    s = jnp.sum(e, axis=-1, keepdims=True)
    lse = jnp.log(s) + m
    onehot = jax.lax.broadcasted_iota(jnp.int32, conf.shape, 1) == ct
    tgt = jnp.sum(jnp.where(onehot, conf, 0.0), axis=-1, keepdims=True)
    ce_ref[0] = lse - tgt

    pos = ct > 0
    d = ld_ref[0] - lt_ref[0]
    ad = jnp.abs(d)
    sl1 = jnp.where(ad < 1.0, 0.5 * d * d, ad - 0.5)
    part = jnp.sum(jnp.where(pos, sl1, 0.0))

    @pl.when(pl.program_id(0) == 0)
    def _():
        ll_ref[...] = jnp.zeros_like(ll_ref)
    ll_ref[...] = ll_ref[...] + part


def _phase1(conf_t, conf_data, loc_t, loc_data):
    B, P = conf_t.shape
    nblk = (B * P) // _TP
    ct3 = conf_t.reshape(nblk, _TP, 1)
    conf3 = conf_data.reshape(nblk, _TP, _C)
    lt3 = loc_t.reshape(nblk, _TP, 4)
    ld3 = loc_data.reshape(nblk, _TP, 4)
    ce, ll = pl.pallas_call(
        _phase1_body,
        grid=(nblk,),
        in_specs=[
            pl.BlockSpec((1, _TP, 1), lambda i: (i, 0, 0)),
            pl.BlockSpec((1, _TP, _C), lambda i: (i, 0, 0)),
            pl.BlockSpec((1, _TP, 4), lambda i: (i, 0, 0)),
            pl.BlockSpec((1, _TP, 4), lambda i: (i, 0, 0)),
        ],
        out_specs=[
            pl.BlockSpec((1, _TP, 1), lambda i: (i, 0, 0)),
            pl.BlockSpec((1, 1), lambda i: (0, 0)),
        ],
        out_shape=[
            jax.ShapeDtypeStruct((nblk, _TP, 1), jnp.float32),
            jax.ShapeDtypeStruct((1, 1), jnp.float32),
        ],
    )(ct3, conf3, lt3, ld3)
    return ce.reshape(B, P), ll[0, 0]


def _phase2_host(conf_t, ce):
    B, P = conf_t.shape
    pos = conf_t > 0
    loss_c = jnp.where(pos, 0.0, ce)
    loss_idx = jnp.argsort(-loss_c, axis=1)
    idx_rank = jnp.argsort(loss_idx, axis=1)
    num_pos = jnp.sum(pos.astype(jnp.int32), axis=1, keepdims=True)
    num_neg = jnp.minimum(3 * num_pos, P - 1)
    neg = idx_rank < num_neg
    mask = jnp.logical_or(pos, neg)
    loss_c_sum = jnp.sum(ce * mask.astype(ce.dtype))
    N = jnp.sum(num_pos).astype(jnp.float32)
    return loss_c_sum, N


def kernel(loc_t, loc_data, conf_t, conf_data):
    ce, loss_l = _phase1(conf_t, conf_data, loc_t, loc_data)
    loss_c_sum, N = _phase2_host(conf_t, ce)
    return loss_l / N, loss_c_sum / N


# speedup vs baseline: 1.8117x; 1.8117x over previous
import functools
import jax
import jax.numpy as jnp
from jax.experimental import pallas as pl

_C = 81
_TP = 2000
_NJ = 10


def _phase1_body(ctT_ref, conf_ref, lt_ref, ld_ref, ce_ref, ll_ref):
    j = pl.program_id(0)
    b = pl.program_id(1)

    conf = conf_ref[0]
    ct32 = ctT_ref[...]
    bsel = jax.lax.broadcasted_iota(jnp.int32, ct32.shape, 1) == b
    ct = jnp.sum(jnp.where(bsel, ct32, 0), axis=1, keepdims=True)

    m = jnp.max(conf)
    e = jnp.exp(conf - m)
    oh = jax.lax.broadcasted_iota(jnp.int32, conf.shape, 1) == ct
    et = jnp.sum(jnp.where(oh, e, 0.0), axis=-1, keepdims=True)
    scaled = e * (1.0 / et)
    ones = jnp.ones((1, _C), jnp.float32)
    s_row = jax.lax.dot_general(ones, scaled, (((1,), (1,)), ((), ())),
                                preferred_element_type=jnp.float32)
    ce_ref[0] = jnp.maximum(jnp.log(s_row), 0.0)

    pos = ct > 0
    d = ld_ref[0] - lt_ref[0]
    ad = jnp.abs(d)
    sl1 = jnp.where(ad < 1.0, 0.5 * d * d, ad - 0.5)
    part = jnp.sum(jnp.where(pos, sl1, 0.0))

    @pl.when(jnp.logical_and(j == 0, b == 0))
    def _():
        ll_ref[...] = jnp.zeros_like(ll_ref)
    ll_ref[...] = ll_ref[...] + part


def _phase1(conf_t, conf_data, loc_t, loc_data):
    B, P = conf_t.shape
    ctT = conf_t.T
    ce, ll = pl.pallas_call(
        _phase1_body,
        grid=(_NJ, B),
        in_specs=[
            pl.BlockSpec((_TP, B), lambda j, b: (j, 0)),
            pl.BlockSpec((1, _TP, _C), lambda j, b: (b, j, 0)),
            pl.BlockSpec((1, _TP, 4), lambda j, b: (b, j, 0)),
            pl.BlockSpec((1, _TP, 4), lambda j, b: (b, j, 0)),
        ],
        out_specs=[
            pl.BlockSpec((1, 1, _TP), lambda j, b: (b * _NJ + j, 0, 0)),
            pl.BlockSpec((1, 1), lambda j, b: (0, 0)),
        ],
        out_shape=[
            jax.ShapeDtypeStruct((B * _NJ, 1, _TP), jnp.float32),
            jax.ShapeDtypeStruct((1, 1), jnp.float32),
        ],
    )(ctT, conf_data, loc_t, loc_data)
    return ce.reshape(B, P), ll[0, 0]


def _phase2_host(conf_t, ce):
    B, P = conf_t.shape
    pos = conf_t > 0
    loss_c = jnp.where(pos, 0.0, ce)
    loss_idx = jnp.argsort(-loss_c, axis=1)
    idx_rank = jnp.argsort(loss_idx, axis=1)
    num_pos = jnp.sum(pos.astype(jnp.int32), axis=1, keepdims=True)
    num_neg = jnp.minimum(3 * num_pos, P - 1)
    neg = idx_rank < num_neg
    mask = jnp.logical_or(pos, neg)
    loss_c_sum = jnp.sum(ce * mask.astype(ce.dtype))
    N = jnp.sum(num_pos).astype(jnp.float32)
    return loss_c_sum, N


def kernel(loc_t, loc_data, conf_t, conf_data):
    ce, loss_l = _phase1(conf_t, conf_data, loc_t, loc_data)
    loss_c_sum, N = _phase2_host(conf_t, ce)
    return loss_l / N, loss_c_sum / N


# speedup vs baseline: 3.4116x; 1.8831x over previous
import functools
import jax
import jax.numpy as jnp
from jax import lax
from jax.experimental import pallas as pl
from jax.experimental.pallas import tpu as pltpu, tpu_sc as plsc

_C = 81
_TP = 2000
_NJ = 10
_P = 20000
_NV = _P // 16


def _phase1_body(ctT_ref, conf_ref, lt_ref, ld_ref,
                 ce_ref, key_ref, ll_ref):
    j = pl.program_id(0)
    b = pl.program_id(1)

    conf = conf_ref[0]
    ct32 = ctT_ref[...]
    bsel = jax.lax.broadcasted_iota(jnp.int32, ct32.shape, 1) == b
    ct = jnp.sum(jnp.where(bsel, ct32, 0), axis=1, keepdims=True)

    m = jnp.max(conf)
    e = jnp.exp(conf - m)
    cls = jax.lax.broadcasted_iota(jnp.int32, conf.shape, 1)
    oh = cls == ct
    et = jnp.sum(jnp.where(oh, e, 0.0), axis=-1, keepdims=True)
    scaled = e * (1.0 / et)
    ones = jnp.ones((1, _C), jnp.float32)
    s_row = jax.lax.dot_general(ones, scaled, (((1,), (1,)), ((), ())),
                                preferred_element_type=jnp.float32)
    ce_row = jnp.maximum(jnp.log(s_row), 0.0)
    ce_ref[0] = ce_row

    posc = jnp.where(jnp.logical_and(oh, cls > 0), 1.0, 0.0)
    pos_row = jax.lax.dot_general(ones, posc, (((1,), (1,)), ((), ())),
                                  preferred_element_type=jnp.float32)
    key_ref[0] = jax.lax.bitcast_convert_type(
        jnp.where(pos_row > 0.5, 0.0, ce_row), jnp.int32)

    pos = ct > 0
    d = ld_ref[0] - lt_ref[0]
    ad = jnp.abs(d)
    sl1 = jnp.where(ad < 1.0, 0.5 * d * d, ad - 0.5)
    part = jnp.sum(jnp.where(pos, sl1, 0.0))

    @pl.when(jnp.logical_and(j == 0, b == 0))
    def _():
        ll_ref[...] = jnp.zeros_like(ll_ref)
    ll_ref[...] = ll_ref[...] + part


def _phase1(conf_t, conf_data, loc_t, loc_data):
    B, P = conf_t.shape
    ctT = conf_t.T
    ce, key, ll = pl.pallas_call(
        _phase1_body,
        grid=(_NJ, B),
        in_specs=[
            pl.BlockSpec((_TP, B), lambda j, b: (j, 0)),
            pl.BlockSpec((1, _TP, _C), lambda j, b: (b, j, 0)),
            pl.BlockSpec((1, _TP, 4), lambda j, b: (b, j, 0)),
            pl.BlockSpec((1, _TP, 4), lambda j, b: (b, j, 0)),
        ],
        out_specs=[
            pl.BlockSpec((1, 1, _TP), lambda j, b: (b * _NJ + j, 0, 0)),
            pl.BlockSpec((1, 1, _TP), lambda j, b: (b * _NJ + j, 0, 0)),
            pl.BlockSpec((1, 1), lambda j, b: (0, 0)),
        ],
        out_shape=[
            jax.ShapeDtypeStruct((B * _NJ, 1, _TP), jnp.float32),
            jax.ShapeDtypeStruct((B * _NJ, 1, _TP), jnp.int32),
            jax.ShapeDtypeStruct((1, 1), jnp.float32),
        ],
    )(ctT, conf_data, loc_t, loc_data)
    return ce.reshape(B, P), key.reshape(B, P), ll[0, 0]


_GDN = lax.GatherDimensionNumbers(
    offset_dims=(), collapsed_slice_dims=(0,), start_index_map=(0,))


def _perm(v, idx):
    return lax.gather(v, idx[:, None], _GDN, slice_sizes=(1,),
                      mode=lax.GatherScatterMode.PROMISE_IN_BOUNDS)


def _rotidx(s):
    return (lax.iota(jnp.int32, 16) + s) & 15


def _tsum(v):
    for s in (1, 2, 4, 8):
        v = v + _perm(v, _rotidx(s))
    return v


def _prefix_excl(v):
    lane = lax.iota(jnp.int32, 16)
    w = v
    for s in (1, 2, 4, 8):
        rot = _perm(w, _rotidx(16 - s))
        w = w + jnp.where(lane >= s, rot, jnp.zeros_like(w))
    return w - v


def _mine_body(ct_hbm, ce_hbm, key_hbm, out_hbm,
               ct_v, ce_v, key_v, part_v):
    wid = lax.axis_index("s") * 2 + lax.axis_index("c")
    pltpu.sync_copy(ct_hbm.at[wid], ct_v)
    pltpu.sync_copy(ce_hbm.at[wid], ce_v)
    pltpu.sync_copy(key_hbm.at[wid], key_v)

    ones_i = jnp.ones((16,), jnp.int32)
    zeros_i = jnp.zeros((16,), jnp.int32)
    lane = lax.iota(jnp.int32, 16)

    def npass(i, acc):
        for u in range(5):
            ct = ct_v[pl.ds(i * 80 + u * 16, 16)]
            acc = acc + jnp.where(ct > 0, ones_i, zeros_i)
        return acc

    npos_vec = _tsum(lax.fori_loop(0, 250, npass, zeros_i))
    k_vec = jnp.minimum(npos_vec * 3, _P - 1)

    prefix = zeros_i
    for b in range(30, -1, -1):
        trial = prefix | (1 << b)

        def cpass(i, acc, trial=trial):
            for u in range(5):
                key = key_v[pl.ds(i * 80 + u * 16, 16)]
                acc = acc + jnp.where(key >= trial, ones_i, zeros_i)
            return acc

        cnt = _tsum(lax.fori_loop(0, 250, cpass, zeros_i))
        prefix = jnp.where(cnt >= k_vec, trial, prefix)
    t_key = prefix

    def gpass(i, acc):
        for u in range(5):
            key = key_v[pl.ds(i * 80 + u * 16, 16)]
            acc = acc + jnp.where(key > t_key, ones_i, zeros_i)
        return acc

    c_gt = _tsum(lax.fori_loop(0, 250, gpass, zeros_i))
    m_vec = k_vec - c_gt

    def ipass(i, carry):
        run, cut = carry
        for u in range(5):
            key = key_v[pl.ds(i * 80 + u * 16, 16)]
            eqi = jnp.where(key == t_key, ones_i, zeros_i)
            pre = _prefix_excl(eqi) + run
            sel = jnp.logical_and(eqi > 0, pre == (m_vec - 1))
            gidx = lane + (i * 80 + u * 16)
            cut = cut + _tsum(jnp.where(sel, gidx + 1, 0))
            run = run + _tsum(eqi)
        return run, cut

    _, cut = lax.fori_loop(0, 250, ipass, (zeros_i, zeros_i))
    idx_cut = jnp.where(m_vec == 0, 0, cut)

    def fin(i, acc):
        for u in range(5):
            key = key_v[pl.ds(i * 80 + u * 16, 16)]
            ce = ce_v[pl.ds(i * 80 + u * 16, 16)]
            gidx = lane + (i * 80 + u * 16)
            take = jnp.logical_and(key == t_key, gidx < idx_cut)
            mask = jnp.logical_or(key == 0,
                                  jnp.logical_or(key > t_key, take))
            acc = acc + jnp.where(mask, ce, 0.0)
        return acc

    acc = lax.fori_loop(0, 250, fin, jnp.zeros((16,), jnp.float32))
    part_v[0] = acc
    part_v[1] = npos_vec.astype(jnp.float32)
    pltpu.sync_copy(part_v, out_hbm.at[wid])


def _phase2_sc(conf_t, ce, key):
    mesh = plsc.VectorSubcoreMesh(core_axis_name="c", subcore_axis_name="s")
    parts = pl.kernel(
        _mine_body,
        out_type=jax.ShapeDtypeStruct((32, 2, 16), jnp.float32),
        mesh=mesh,
        scratch_types=[
            pltpu.VMEM((_P,), jnp.int32),
            pltpu.VMEM((_P,), jnp.float32),
            pltpu.VMEM((_P,), jnp.int32),
            pltpu.VMEM((2, 16), jnp.float32),
        ],
    )(conf_t, ce, key)
    return jnp.sum(parts[:, 0, :]), jnp.sum(parts[:, 1, :]) / 16.0


def kernel(loc_t, loc_data, conf_t, conf_data):
    ce, key, loss_l = _phase1(conf_t, conf_data, loc_t, loc_data)
    loss_c_sum, N = _phase2_sc(conf_t, ce, key)
    return loss_l / N, loss_c_sum / N
